# baseline (speedup 1.0000x reference)
import functools

import jax
import jax.numpy as jnp
from jax import lax
from jax.experimental import pallas as pl
from jax.experimental.pallas import tpu as pltpu
from jax.experimental.pallas import tpu_sc as plsc

_N_SPARSE = 26
_N_DENSE = 13
_VOCAB = 100000
_EMB = 16
_B = 16384
_FH = _N_SPARSE // 2
_SLOTS = 16
_GW = _SLOTS * _EMB
_ROWS_H = _B * _SLOTS
_VSEG = _VOCAB // 8

_NC, _NS = 2, 16
_NW = _NC * _NS
_RPW = _ROWS_H // _NW
_CHUNK = 1024
_NCHUNK = _RPW // _CHUNK


def _sc_gather(tables_half, idx_flat):
  mesh = plsc.VectorSubcoreMesh(core_axis_name="c", subcore_axis_name="s")

  @functools.partial(
      pl.kernel,
      mesh=mesh,
      out_type=jax.ShapeDtypeStruct((_ROWS_H, _EMB), jnp.float32),
      scratch_types=[
          pltpu.VMEM((_CHUNK,), jnp.int32),
          pltpu.VMEM((_CHUNK, _EMB), jnp.float32),
          pltpu.SemaphoreType.DMA,
      ],
      compiler_params=pltpu.CompilerParams(use_tc_tiling_on_sc=False),
  )
  def k(tab_hbm, idx_hbm, out_hbm, idx_v, rows_v, sem):
    wid = lax.axis_index("s") * _NC + lax.axis_index("c")
    base = wid * _RPW
    for j in range(_NCHUNK):
      off = base + j * _CHUNK
      pltpu.sync_copy(idx_hbm.at[pl.ds(off, _CHUNK)], idx_v)
      pltpu.make_async_copy(tab_hbm.at[idx_v], rows_v, sem).start()
      pltpu.make_async_copy(tab_hbm.at[idx_v], rows_v, sem).wait()
      pltpu.sync_copy(rows_v, out_hbm.at[pl.ds(off, _CHUNK)])

  return k(tables_half, idx_flat)


def _tr_body(in_ref, out_hbm, x_scr, y_scr, sem):
  f = pl.program_id(0)
  nf = pl.num_programs(0)
  for j in range(8):
    x_scr[j * _EMB:(j + 1) * _EMB, :] = in_ref[0, :, j * _VSEG:(j + 1) * _VSEG]
  off = jax.lax.rem(f, 2) * _VSEG

  @pl.when(f >= 2)
  def _wait_slot():
    pltpu.make_async_copy(
        y_scr.at[pl.ds(off, _VSEG)],
        out_hbm.at[pl.ds((f - 2) * _VSEG, _VSEG)], sem).wait()

  y_scr[pl.ds(off, _VSEG), :] = x_scr[...].T
  pltpu.make_async_copy(
      y_scr.at[pl.ds(off, _VSEG)],
      out_hbm.at[pl.ds(f * _VSEG, _VSEG)], sem).start()

  @pl.when(f == nf - 1)
  def _drain_all():
    for _ in range(2):
      pltpu.make_async_copy(
          y_scr.at[pl.ds(off, _VSEG)],
          out_hbm.at[pl.ds(f * _VSEG, _VSEG)], sem).wait()


def _tc_transpose(tphys):
  return pl.pallas_call(
      _tr_body,
      grid=(_FH,),
      in_specs=[pl.BlockSpec((1, _EMB, _VOCAB), lambda f: (f, 0, 0))],
      out_specs=pl.BlockSpec(memory_space=pl.ANY),
      out_shape=jax.ShapeDtypeStruct((_FH * _VSEG, 128), jnp.float32),
      scratch_shapes=[
          pltpu.VMEM((128, _VSEG), jnp.float32),
          pltpu.VMEM((2 * _VSEG, 128), jnp.float32),
          pltpu.SemaphoreType.DMA,
      ],
      compiler_params=pltpu.CompilerParams(
          dimension_semantics=("arbitrary",)),
  )(tphys)


def _dnn_body(g1_ref, g2_ref, d_ref, s_ref, m_ref, w1p1_ref, w1p2_ref,
              w1b_ref, b1_ref, w2_ref, b2_ref, w3_ref, b3_ref, wf_ref,
              bf_ref, out_ref):
  f32 = jnp.float32
  g1 = g1_ref[...]
  g2 = g2_ref[...]
  dd = d_ref[...]
  m = m_ref[...]
  sum_e = (lax.dot(g1, s_ref[...], preferred_element_type=f32)
           + lax.dot(g2, s_ref[...], preferred_element_type=f32))
  t1 = jnp.sum(sum_e * sum_e, axis=1, keepdims=True)
  t2 = (jnp.sum(g1 * g1 * m, axis=1, keepdims=True)
        + jnp.sum(g2 * g2 * m, axis=1, keepdims=True))
  wide = 0.5 * (t1 - t2)
  h = lax.dot(g1, w1p1_ref[...], preferred_element_type=f32)
  h = h + lax.dot(g2, w1p2_ref[...], preferred_element_type=f32)
  h = h + lax.dot(dd, w1b_ref[...], preferred_element_type=f32)
  h = jax.nn.relu(h + b1_ref[...])
  h = jax.nn.relu(lax.dot(h, w2_ref[...], preferred_element_type=f32)
                  + b2_ref[...])
  h = jax.nn.relu(lax.dot(h, w3_ref[...], preferred_element_type=f32)
                  + b3_ref[...])
  z = lax.dot(wide + h, wf_ref[...], preferred_element_type=f32) + bf_ref[...]
  out_ref[...] = jax.nn.sigmoid(z)


_BB = 512


def _dnn(g1, g2, dense, s, m, w1p1, w1p2, w1b, b1, w2, b2, w3, b3, wf, bf):
  def row_block(i):
    return (i, 0)

  def full(i):
    return (0, 0)

  return pl.pallas_call(
      _dnn_body,
      grid=(_B // _BB,),
      in_specs=[
          pl.BlockSpec((_BB, _GW), row_block),
          pl.BlockSpec((_BB, _GW), row_block),
          pl.BlockSpec((_BB, _N_DENSE), row_block),
          pl.BlockSpec((_GW, _EMB), full),
          pl.BlockSpec((1, _GW), full),
          pl.BlockSpec((_GW, 256), full),
          pl.BlockSpec((_GW, 256), full),
          pl.BlockSpec((_N_DENSE, 256), full),
          pl.BlockSpec((1, 256), full),
          pl.BlockSpec((256, 128), full),
          pl.BlockSpec((1, 128), full),
          pl.BlockSpec((128, 64), full),
          pl.BlockSpec((1, 64), full),
          pl.BlockSpec((64, 1), full),
          pl.BlockSpec((1, 1), full),
      ],
      out_specs=pl.BlockSpec((_BB, 1), row_block),
      out_shape=jax.ShapeDtypeStruct((_B, 1), jnp.float32),
      compiler_params=pltpu.CompilerParams(
          dimension_semantics=("parallel",)),
  )(g1, g2, dense, s, m, w1p1, w1p2, w1b, b1, w2, b2, w3, b3, wf, bf)


def _half_idx(sparse_half):
  perm = (jnp.arange(_FH, dtype=jnp.int32)[None, :] * _VSEG
          + sparse_half % _VSEG) * 8 + sparse_half // _VSEG
  idx = jnp.concatenate([perm, perm[:, :_SLOTS - _FH]], axis=1)
  return idx.reshape(-1)


def kernel(x, tables, W1, b1, W2, b2, W3, b3, Wf, bf):
  sparse_idx = x[:, :_N_SPARSE].astype(jnp.int32)
  dense = x[:, _N_SPARSE:]
  idx1 = _half_idx(sparse_idx[:, :_FH])
  idx2 = _half_idx(sparse_idx[:, _FH:])
  tphys = jnp.transpose(tables, (0, 2, 1))
  t1 = _tc_transpose(tphys[:_FH])
  th1 = t1.reshape(_FH * _VOCAB, _EMB)
  g1 = _sc_gather(th1, idx1).reshape(_B, _GW)
  t2 = _tc_transpose(tphys[_FH:])
  th2 = t2.reshape(_FH * _VOCAB, _EMB)
  g2 = _sc_gather(th2, idx2).reshape(_B, _GW)
  pad = _GW - _FH * _EMB
  s = jnp.concatenate(
      [jnp.tile(jnp.eye(_EMB, dtype=jnp.float32), (_FH, 1)),
       jnp.zeros((pad, _EMB), jnp.float32)], axis=0)
  m = (jnp.arange(_GW, dtype=jnp.float32)
       < _FH * _EMB).astype(jnp.float32)[None, :]
  zpad = jnp.zeros((pad, W1.shape[1]), jnp.float32)
  w1p1 = jnp.concatenate([W1[:_FH * _EMB], zpad], axis=0)
  w1p2 = jnp.concatenate([W1[_FH * _EMB:2 * _FH * _EMB], zpad], axis=0)
  return _dnn(g1, g2, dense, s, m, w1p1, w1p2, W1[2 * _FH * _EMB:],
              b1.reshape(1, -1), W2, b2.reshape(1, -1), W3,
              b3.reshape(1, -1), Wf, bf.reshape(1, 1))
